# baseline (speedup 1.0000x reference)
import functools

import jax
import jax.numpy as jnp
from jax import lax
from jax.experimental import pallas as pl
from jax.experimental.pallas import tpu as pltpu
from jax.experimental.pallas import tpu_sc as plsc

N = 10000
E = 320000
D = 128

NC, NS = 2, 16
NW = NC * NS
CHUNK = 128
NCHUNK = 2560
ITERS = NCHUNK // NW
EPAD = NCHUNK * CHUNK
NPAD = 10240
DEG_PER_TILE = NPAD // NS
ROWS_PER_TILE = NPAD // NS

_mesh = plsc.VectorSubcoreMesh(core_axis_name="c", subcore_axis_name="s")


@functools.partial(
    pl.kernel,
    out_type=jax.ShapeDtypeStruct((NC * NPAD,), jnp.float32),
    mesh=_mesh,
    scratch_types=[
        pltpu.VMEM((ITERS, CHUNK), jnp.int32),
        pltpu.VMEM((CHUNK,), jnp.float32),
        pltpu.VMEM((DEG_PER_TILE,), jnp.float32),
        pltpu.VMEM_SHARED((NPAD,), jnp.float32),
        pltpu.SemaphoreType.DMA,
    ],
)
def _deg_kernel(dst_hbm, out_hbm, di_v, ones_v, buf_v, acc_sh, sem):
    del sem
    cid = lax.axis_index("c")
    sid = lax.axis_index("s")
    wid = sid * NC + cid

    for j in range(CHUNK // 16):
        ones_v[pl.ds(j * 16, 16)] = jnp.ones((16,), jnp.float32)

    def _zero(i, carry):
        buf_v[pl.ds(i * 16, 16)] = jnp.zeros((16,), jnp.float32)
        return carry

    lax.fori_loop(0, DEG_PER_TILE // 16, _zero, 0)
    pltpu.sync_copy(buf_v, acc_sh.at[pl.ds(sid * DEG_PER_TILE, DEG_PER_TILE)])
    plsc.subcore_barrier()

    pltpu.sync_copy(dst_hbm.at[pl.ds(wid * ITERS, ITERS)], di_v)

    def _body(j, carry):
        pltpu.sync_copy(ones_v, acc_sh.at[di_v.at[j]], add=True)
        return carry

    lax.fori_loop(0, ITERS, _body, 0)
    plsc.subcore_barrier()

    pltpu.sync_copy(acc_sh.at[pl.ds(sid * DEG_PER_TILE, DEG_PER_TILE)], buf_v)
    pltpu.sync_copy(
        buf_v, out_hbm.at[pl.ds(cid * NPAD + sid * DEG_PER_TILE, DEG_PER_TILE)]
    )


@functools.partial(
    pl.kernel,
    out_type=jax.ShapeDtypeStruct((NC * NPAD, D), jnp.float32),
    mesh=_mesh,
    scratch_types=[
        pltpu.VMEM((ITERS, CHUNK), jnp.int32),
        pltpu.VMEM((ITERS, CHUNK), jnp.int32),
        pltpu.VMEM((CHUNK,), jnp.int32),
        pltpu.VMEM((CHUNK,), jnp.int32),
        pltpu.VMEM((CHUNK, D), jnp.float32),
        pltpu.VMEM_SHARED((NPAD, D), jnp.float32),
        pltpu.SemaphoreType.DMA,
    ],
)
def _scatter_kernel(src_hbm, dst_hbm, hp_hbm, out_hbm, si_v, di_v, si1_v,
                    di1_v, rows_v, acc_sh, sem):
    cid = lax.axis_index("c")
    sid = lax.axis_index("s")
    wid = sid * NC + cid

    def _zero(i, carry):
        for j in range(D // 16):
            rows_v[i, pl.ds(j * 16, 16)] = jnp.zeros((16,), jnp.float32)
        return carry

    lax.fori_loop(0, CHUNK, _zero, 0)

    r0 = sid * ROWS_PER_TILE
    for t in range(ROWS_PER_TILE // CHUNK):
        pltpu.sync_copy(rows_v, acc_sh.at[pl.ds(r0 + t * CHUNK, CHUNK)])
    plsc.subcore_barrier()

    pltpu.sync_copy(src_hbm.at[pl.ds(wid * ITERS, ITERS)], si_v)
    pltpu.sync_copy(dst_hbm.at[pl.ds(wid * ITERS, ITERS)], di_v)

    def _body(j, carry):
        for k in range(CHUNK // 16):
            si1_v[pl.ds(k * 16, 16)] = si_v[j, pl.ds(k * 16, 16)]
            di1_v[pl.ds(k * 16, 16)] = di_v[j, pl.ds(k * 16, 16)]
        pltpu.async_copy(hp_hbm.at[si1_v], rows_v, sem).wait()
        pltpu.sync_copy(rows_v, acc_sh.at[di1_v], add=True)
        return carry

    lax.fori_loop(0, ITERS, _body, 0)

    plsc.subcore_barrier()

    for t in range(ROWS_PER_TILE // CHUNK):
        pltpu.sync_copy(acc_sh.at[pl.ds(r0 + t * CHUNK, CHUNK)], rows_v)
        pltpu.sync_copy(rows_v,
                        out_hbm.at[pl.ds(cid * NPAD + r0 + t * CHUNK, CHUNK)])


MB = 1000


def _mm_body(x_ref, w_ref, deg_ref, hp_ref, dis_ref):
    dis = lax.rsqrt(deg_ref[...])
    h = jnp.dot(x_ref[...], w_ref[...], preferred_element_type=jnp.float32)
    hp_ref[...] = h * dis
    dis_ref[...] = dis


_mm_call = pl.pallas_call(
    _mm_body,
    grid=(N // MB,),
    in_specs=[
        pl.BlockSpec((MB, D), lambda i: (i, 0)),
        pl.BlockSpec((D, D), lambda i: (0, 0)),
        pl.BlockSpec((MB, 1), lambda i: (i, 0)),
    ],
    out_specs=[
        pl.BlockSpec((MB, D), lambda i: (i, 0)),
        pl.BlockSpec((MB, 1), lambda i: (i, 0)),
    ],
    out_shape=[
        jax.ShapeDtypeStruct((N, D), jnp.float32),
        jax.ShapeDtypeStruct((N, 1), jnp.float32),
    ],
)


def _final_body(acc_ref, hp_ref, dis_ref, b_ref, o_ref):
    s = acc_ref[0] + acc_ref[1] + hp_ref[...]
    o_ref[...] = jnp.maximum(s * dis_ref[...] + b_ref[...], 0.0)


_final_call = pl.pallas_call(
    _final_body,
    grid=(N // MB,),
    in_specs=[
        pl.BlockSpec((NC, MB, D), lambda i: (0, i, 0)),
        pl.BlockSpec((MB, D), lambda i: (i, 0)),
        pl.BlockSpec((MB, 1), lambda i: (i, 0)),
        pl.BlockSpec((1, D), lambda i: (0, 0)),
    ],
    out_specs=pl.BlockSpec((MB, D), lambda i: (i, 0)),
    out_shape=jax.ShapeDtypeStruct((N, D), jnp.float32),
)


@jax.jit
def kernel(x, edge_index, W, b):
    src = edge_index[0].astype(jnp.int32)
    dst = edge_index[1].astype(jnp.int32)

    npad_e = EPAD - E
    srcp = jnp.concatenate(
        [src, jnp.zeros((npad_e,), jnp.int32)]).reshape(NCHUNK, CHUNK)
    dstp = jnp.concatenate(
        [dst, jnp.full((npad_e,), NPAD - 1, jnp.int32)]).reshape(NCHUNK, CHUNK)

    degp = _deg_kernel(dstp)
    deg = (1.0 + degp[:N] + degp[NPAD:NPAD + N]).reshape(N, 1)

    hp, dis = _mm_call(x, W, deg)

    acc = _scatter_kernel(srcp, dstp, hp).reshape(NC, NPAD, D)

    return _final_call(acc, hp, dis, b.reshape(1, D))


# speedup vs baseline: 2.4308x; 2.4308x over previous
import functools

import jax
import jax.numpy as jnp
from jax import lax
from jax.experimental import pallas as pl
from jax.experimental.pallas import tpu as pltpu
from jax.experimental.pallas import tpu_sc as plsc

N = 10000
E = 320000
D = 128

NC, NS = 2, 16
NW = NC * NS
CHUNK = 128
NCHUNK = 2560
ITERS = NCHUNK // NW
EPAD = NCHUNK * CHUNK
NPAD = 10240
DEG_PER_TILE = NPAD // NS
ROWS_PER_TILE = NPAD // NS

_mesh = plsc.VectorSubcoreMesh(core_axis_name="c", subcore_axis_name="s")


@functools.partial(
    pl.kernel,
    out_type=jax.ShapeDtypeStruct((NC * NPAD,), jnp.float32),
    mesh=_mesh,
    scratch_types=[
        pltpu.VMEM((ITERS, CHUNK), jnp.int32),
        pltpu.VMEM((CHUNK,), jnp.float32),
        pltpu.VMEM((DEG_PER_TILE,), jnp.float32),
        pltpu.VMEM_SHARED((NPAD,), jnp.float32),
        pltpu.SemaphoreType.DMA,
    ],
)
def _deg_kernel(dst_hbm, out_hbm, di_v, ones_v, buf_v, acc_sh, sem):
    del sem
    cid = lax.axis_index("c")
    sid = lax.axis_index("s")
    wid = sid * NC + cid

    for j in range(CHUNK // 16):
        ones_v[pl.ds(j * 16, 16)] = jnp.ones((16,), jnp.float32)

    def _zero(i, carry):
        buf_v[pl.ds(i * 16, 16)] = jnp.zeros((16,), jnp.float32)
        return carry

    lax.fori_loop(0, DEG_PER_TILE // 16, _zero, 0)
    pltpu.sync_copy(buf_v, acc_sh.at[pl.ds(sid * DEG_PER_TILE, DEG_PER_TILE)])
    plsc.subcore_barrier()

    pltpu.sync_copy(dst_hbm.at[pl.ds(wid * ITERS, ITERS)], di_v)

    def _body(j, carry):
        pltpu.sync_copy(ones_v, acc_sh.at[di_v.at[j]], add=True)
        return carry

    lax.fori_loop(0, ITERS, _body, 0)
    plsc.subcore_barrier()

    pltpu.sync_copy(acc_sh.at[pl.ds(sid * DEG_PER_TILE, DEG_PER_TILE)], buf_v)
    pltpu.sync_copy(
        buf_v, out_hbm.at[pl.ds(cid * NPAD + sid * DEG_PER_TILE, DEG_PER_TILE)]
    )


@functools.partial(
    pl.kernel,
    out_type=jax.ShapeDtypeStruct((NC * NPAD, D), jnp.float32),
    mesh=_mesh,
    scratch_types=[
        pltpu.VMEM((ITERS, CHUNK), jnp.int32),
        pltpu.VMEM((ITERS, CHUNK), jnp.int32),
        pltpu.VMEM((CHUNK,), jnp.int32),
        pltpu.VMEM((CHUNK,), jnp.int32),
        pltpu.VMEM((CHUNK, D), jnp.float32),
        pltpu.VMEM_SHARED((NPAD, D), jnp.float32),
        pltpu.SemaphoreType.DMA,
    ],
)
def _scatter_kernel(src_hbm, dst_hbm, hp_hbm, out_hbm, si_v, di_v, si1_v,
                    di1_v, rows_v, acc_sh, sem):
    cid = lax.axis_index("c")
    sid = lax.axis_index("s")
    wid = sid * NC + cid

    def _zero(i, carry):
        for j in range(D // 16):
            rows_v[i, pl.ds(j * 16, 16)] = jnp.zeros((16,), jnp.float32)
        return carry

    lax.fori_loop(0, CHUNK, _zero, 0)

    r0 = sid * ROWS_PER_TILE
    for t in range(ROWS_PER_TILE // CHUNK):
        pltpu.sync_copy(rows_v, acc_sh.at[pl.ds(r0 + t * CHUNK, CHUNK)])
    plsc.subcore_barrier()

    pltpu.sync_copy(src_hbm.at[pl.ds(wid * ITERS, ITERS)], si_v)
    pltpu.sync_copy(dst_hbm.at[pl.ds(wid * ITERS, ITERS)], di_v)

    def _body(j, carry):
        for k in range(CHUNK // 16):
            si1_v[pl.ds(k * 16, 16)] = si_v[j, pl.ds(k * 16, 16)]
            di1_v[pl.ds(k * 16, 16)] = di_v[j, pl.ds(k * 16, 16)]
        pltpu.async_copy(hp_hbm.at[si1_v], rows_v, sem).wait()
        pltpu.sync_copy(rows_v, acc_sh.at[di1_v], add=True)
        return carry

    lax.fori_loop(0, ITERS, _body, 0)

    plsc.subcore_barrier()

    for t in range(ROWS_PER_TILE // CHUNK):
        pltpu.sync_copy(acc_sh.at[pl.ds(r0 + t * CHUNK, CHUNK)], rows_v)
        pltpu.sync_copy(rows_v,
                        out_hbm.at[pl.ds(cid * NPAD + r0 + t * CHUNK, CHUNK)])


MB = 1000


def _mm_body(x_ref, w_ref, deg_ref, hp_ref, dis_ref):
    dis = lax.rsqrt(deg_ref[...])
    h = jnp.dot(x_ref[...], w_ref[...], preferred_element_type=jnp.float32)
    hp_ref[...] = h * dis
    dis_ref[...] = dis


_mm_call = pl.pallas_call(
    _mm_body,
    grid=(N // MB,),
    in_specs=[
        pl.BlockSpec((MB, D), lambda i: (i, 0)),
        pl.BlockSpec((D, D), lambda i: (0, 0)),
        pl.BlockSpec((MB, 1), lambda i: (i, 0)),
    ],
    out_specs=[
        pl.BlockSpec((MB, D), lambda i: (i, 0)),
        pl.BlockSpec((MB, 1), lambda i: (i, 0)),
    ],
    out_shape=[
        jax.ShapeDtypeStruct((N, D), jnp.float32),
        jax.ShapeDtypeStruct((N, 1), jnp.float32),
    ],
)


def _final_body(acc_ref, hp_ref, dis_ref, b_ref, o_ref):
    s = acc_ref[0] + acc_ref[1] + hp_ref[...]
    o_ref[...] = jnp.maximum(s * dis_ref[...] + b_ref[...], 0.0)


_final_call = pl.pallas_call(
    _final_body,
    grid=(N // MB,),
    in_specs=[
        pl.BlockSpec((NC, MB, D), lambda i: (0, i, 0)),
        pl.BlockSpec((MB, D), lambda i: (i, 0)),
        pl.BlockSpec((MB, 1), lambda i: (i, 0)),
        pl.BlockSpec((1, D), lambda i: (0, 0)),
    ],
    out_specs=pl.BlockSpec((MB, D), lambda i: (i, 0)),
    out_shape=jax.ShapeDtypeStruct((N, D), jnp.float32),
)


@jax.jit
def kernel(x, edge_index, W, b):
    src = edge_index[0].astype(jnp.int32)
    dst = edge_index[1].astype(jnp.int32)

    npad_e = EPAD - E
    pad_iota = jnp.arange(npad_e, dtype=jnp.int32)
    srcp = jnp.concatenate([src, pad_iota % N]).reshape(NCHUNK, CHUNK)
    dstp = jnp.concatenate(
        [dst, N + pad_iota % (NPAD - N)]).reshape(NCHUNK, CHUNK)

    degp = _deg_kernel(dstp)
    deg = (1.0 + degp[:N] + degp[NPAD:NPAD + N]).reshape(N, 1)

    hp, dis = _mm_call(x, W, deg)

    acc = _scatter_kernel(srcp, dstp, hp).reshape(NC, NPAD, D)

    return _final_call(acc, hp, dis, b.reshape(1, D))


# speedup vs baseline: 3.0282x; 1.2458x over previous
import functools

import jax
import jax.numpy as jnp
from jax import lax
from jax.experimental import pallas as pl
from jax.experimental.pallas import tpu as pltpu
from jax.experimental.pallas import tpu_sc as plsc

N = 10000
E = 320000
D = 128

NC, NS = 2, 16
NW = NC * NS
CHUNK = 128
NCHUNK = 2560
ITERS = NCHUNK // NW
EPAD = NCHUNK * CHUNK
NPAD = 10240
DEG_PER_TILE = NPAD // NS
ROWS_PER_TILE = NPAD // NS

_mesh = plsc.VectorSubcoreMesh(core_axis_name="c", subcore_axis_name="s")


@functools.partial(
    pl.kernel,
    out_type=jax.ShapeDtypeStruct((NC * NPAD,), jnp.float32),
    mesh=_mesh,
    scratch_types=[
        pltpu.VMEM((ITERS, CHUNK), jnp.int32),
        pltpu.VMEM((CHUNK,), jnp.float32),
        pltpu.VMEM((DEG_PER_TILE,), jnp.float32),
        pltpu.VMEM_SHARED((NPAD,), jnp.float32),
        pltpu.SemaphoreType.DMA,
    ],
)
def _deg_kernel(dst_hbm, out_hbm, di_v, ones_v, buf_v, acc_sh, sem):
    del sem
    cid = lax.axis_index("c")
    sid = lax.axis_index("s")
    wid = sid * NC + cid

    for j in range(CHUNK // 16):
        ones_v[pl.ds(j * 16, 16)] = jnp.ones((16,), jnp.float32)

    def _zero(i, carry):
        buf_v[pl.ds(i * 16, 16)] = jnp.zeros((16,), jnp.float32)
        return carry

    lax.fori_loop(0, DEG_PER_TILE // 16, _zero, 0)
    pltpu.sync_copy(buf_v, acc_sh.at[pl.ds(sid * DEG_PER_TILE, DEG_PER_TILE)])
    plsc.subcore_barrier()

    pltpu.sync_copy(dst_hbm.at[pl.ds(wid * ITERS, ITERS)], di_v)

    def _body(j, carry):
        pltpu.sync_copy(ones_v, acc_sh.at[di_v.at[j]], add=True)
        return carry

    lax.fori_loop(0, ITERS, _body, 0)
    plsc.subcore_barrier()

    pltpu.sync_copy(acc_sh.at[pl.ds(sid * DEG_PER_TILE, DEG_PER_TILE)], buf_v)
    pltpu.sync_copy(
        buf_v, out_hbm.at[pl.ds(cid * NPAD + sid * DEG_PER_TILE, DEG_PER_TILE)]
    )


IBLK = ITERS // 2


@functools.partial(
    pl.kernel,
    out_type=jax.ShapeDtypeStruct((NC * NPAD, D), jnp.float32),
    mesh=_mesh,
    scratch_types=[
        pltpu.VMEM((IBLK, CHUNK), jnp.int32),
        pltpu.VMEM((IBLK, CHUNK), jnp.int32),
        pltpu.VMEM((CHUNK, D), jnp.float32),
        pltpu.VMEM((CHUNK, D), jnp.float32),
        pltpu.VMEM_SHARED((NPAD, D), jnp.float32),
        pltpu.SemaphoreType.DMA,
        pltpu.SemaphoreType.DMA,
    ],
)
def _scatter_kernel(src_hbm, dst_hbm, hp_hbm, out_hbm, si_v, di_v, rows0_v,
                    rows1_v, acc_sh, sem0, sem1):
    cid = lax.axis_index("c")
    sid = lax.axis_index("s")
    wid = sid * NC + cid

    def _zero(i, carry):
        for j in range(D // 16):
            rows0_v[i, pl.ds(j * 16, 16)] = jnp.zeros((16,), jnp.float32)
        return carry

    lax.fori_loop(0, CHUNK, _zero, 0)

    r0 = sid * ROWS_PER_TILE
    for t in range(ROWS_PER_TILE // CHUNK):
        pltpu.sync_copy(rows0_v, acc_sh.at[pl.ds(r0 + t * CHUNK, CHUNK)])
    plsc.subcore_barrier()

    for h in range(2):
        base = wid * ITERS + h * IBLK
        pltpu.sync_copy(src_hbm.at[pl.ds(base, IBLK)], si_v)
        pltpu.sync_copy(dst_hbm.at[pl.ds(base, IBLK)], di_v)

        pltpu.async_copy(hp_hbm.at[si_v.at[0]], rows0_v, sem0)

        def _body(g, carry):
            j0 = 2 * g
            j1 = j0 + 1
            pltpu.make_async_copy(hp_hbm.at[si_v.at[j0]], rows0_v, sem0).wait()
            pltpu.async_copy(hp_hbm.at[si_v.at[j1]], rows1_v, sem1)
            pltpu.sync_copy(rows0_v, acc_sh.at[di_v.at[j0]], add=True)
            pltpu.make_async_copy(hp_hbm.at[si_v.at[j1]], rows1_v, sem1).wait()

            @pl.when(g < IBLK // 2 - 1)
            def _():
                pltpu.async_copy(hp_hbm.at[si_v.at[j0 + 2]], rows0_v, sem0)

            pltpu.sync_copy(rows1_v, acc_sh.at[di_v.at[j1]], add=True)
            return carry

        lax.fori_loop(0, IBLK // 2, _body, 0)

    plsc.subcore_barrier()

    for t in range(ROWS_PER_TILE // CHUNK):
        pltpu.sync_copy(acc_sh.at[pl.ds(r0 + t * CHUNK, CHUNK)], rows0_v)
        pltpu.sync_copy(rows0_v,
                        out_hbm.at[pl.ds(cid * NPAD + r0 + t * CHUNK, CHUNK)])


MB = 1000


def _mm_body(x_ref, w_ref, deg_ref, hp_ref, dis_ref):
    dis = lax.rsqrt(deg_ref[...])
    h = jnp.dot(x_ref[...], w_ref[...], preferred_element_type=jnp.float32)
    hp_ref[...] = h * dis
    dis_ref[...] = dis


_mm_call = pl.pallas_call(
    _mm_body,
    grid=(N // MB,),
    in_specs=[
        pl.BlockSpec((MB, D), lambda i: (i, 0)),
        pl.BlockSpec((D, D), lambda i: (0, 0)),
        pl.BlockSpec((MB, 1), lambda i: (i, 0)),
    ],
    out_specs=[
        pl.BlockSpec((MB, D), lambda i: (i, 0)),
        pl.BlockSpec((MB, 1), lambda i: (i, 0)),
    ],
    out_shape=[
        jax.ShapeDtypeStruct((N, D), jnp.float32),
        jax.ShapeDtypeStruct((N, 1), jnp.float32),
    ],
)


def _final_body(acc_ref, hp_ref, dis_ref, b_ref, o_ref):
    s = acc_ref[0] + acc_ref[1] + hp_ref[...]
    o_ref[...] = jnp.maximum(s * dis_ref[...] + b_ref[...], 0.0)


_final_call = pl.pallas_call(
    _final_body,
    grid=(N // MB,),
    in_specs=[
        pl.BlockSpec((NC, MB, D), lambda i: (0, i, 0)),
        pl.BlockSpec((MB, D), lambda i: (i, 0)),
        pl.BlockSpec((MB, 1), lambda i: (i, 0)),
        pl.BlockSpec((1, D), lambda i: (0, 0)),
    ],
    out_specs=pl.BlockSpec((MB, D), lambda i: (i, 0)),
    out_shape=jax.ShapeDtypeStruct((N, D), jnp.float32),
)


@jax.jit
def kernel(x, edge_index, W, b):
    src = edge_index[0].astype(jnp.int32)
    dst = edge_index[1].astype(jnp.int32)

    npad_e = EPAD - E
    pad_iota = jnp.arange(npad_e, dtype=jnp.int32)
    srcp = jnp.concatenate([src, pad_iota % N]).reshape(NCHUNK, CHUNK)
    dstp = jnp.concatenate(
        [dst, N + pad_iota % (NPAD - N)]).reshape(NCHUNK, CHUNK)

    degp = _deg_kernel(dstp)
    deg = (1.0 + degp[:N] + degp[NPAD:NPAD + N]).reshape(N, 1)

    hp, dis = _mm_call(x, W, deg)

    acc = _scatter_kernel(srcp, dstp, hp).reshape(NC, NPAD, D)

    return _final_call(acc, hp, dis, b.reshape(1, D))


# speedup vs baseline: 3.5163x; 1.1612x over previous
import functools

import jax
import jax.numpy as jnp
from jax import lax
from jax.experimental import pallas as pl
from jax.experimental.pallas import tpu as pltpu
from jax.experimental.pallas import tpu_sc as plsc

N = 10000
E = 320000
D = 128

NC, NS = 2, 16
NW = NC * NS
CHUNK = 128
NCHUNK = 2560
ITERS = NCHUNK // NW
EPAD = NCHUNK * CHUNK
NPAD = 10240
DEG_PER_TILE = NPAD // NS
ROWS_PER_TILE = NPAD // NS

_mesh = plsc.VectorSubcoreMesh(core_axis_name="c", subcore_axis_name="s")


@functools.partial(
    pl.kernel,
    out_type=jax.ShapeDtypeStruct((NC * NPAD,), jnp.float32),
    mesh=_mesh,
    scratch_types=[
        pltpu.VMEM((ITERS, CHUNK), jnp.int32),
        pltpu.VMEM((CHUNK,), jnp.float32),
        pltpu.VMEM((DEG_PER_TILE,), jnp.float32),
        pltpu.VMEM_SHARED((NPAD,), jnp.float32),
        pltpu.SemaphoreType.DMA,
    ],
)
def _deg_kernel(dst_hbm, out_hbm, di_v, ones_v, buf_v, acc_sh, sem):
    del sem
    cid = lax.axis_index("c")
    sid = lax.axis_index("s")
    wid = sid * NC + cid

    for j in range(CHUNK // 16):
        ones_v[pl.ds(j * 16, 16)] = jnp.ones((16,), jnp.float32)

    def _zero(i, carry):
        buf_v[pl.ds(i * 16, 16)] = jnp.zeros((16,), jnp.float32)
        return carry

    lax.fori_loop(0, DEG_PER_TILE // 16, _zero, 0)
    pltpu.sync_copy(buf_v, acc_sh.at[pl.ds(sid * DEG_PER_TILE, DEG_PER_TILE)])
    plsc.subcore_barrier()

    pltpu.sync_copy(dst_hbm.at[pl.ds(wid * ITERS, ITERS)], di_v)

    def _body(j, carry):
        pltpu.sync_copy(ones_v, acc_sh.at[di_v.at[j]], add=True)
        return carry

    lax.fori_loop(0, ITERS, _body, 0)
    plsc.subcore_barrier()

    pltpu.sync_copy(acc_sh.at[pl.ds(sid * DEG_PER_TILE, DEG_PER_TILE)], buf_v)
    pltpu.sync_copy(
        buf_v, out_hbm.at[pl.ds(cid * NPAD + sid * DEG_PER_TILE, DEG_PER_TILE)]
    )


CC = 64
ITERS_C = EPAD // CC // NW
IBLK = ITERS_C // 4
NBUF = 4


@functools.partial(
    pl.kernel,
    out_type=jax.ShapeDtypeStruct((NC * NPAD, D), jnp.float32),
    mesh=_mesh,
    scratch_types=[
        pltpu.VMEM((IBLK, CC), jnp.int32),
        pltpu.VMEM((IBLK, CC), jnp.int32),
        pltpu.VMEM((CC, D), jnp.float32),
        pltpu.VMEM((CC, D), jnp.float32),
        pltpu.VMEM((CC, D), jnp.float32),
        pltpu.VMEM((CC, D), jnp.float32),
        pltpu.VMEM_SHARED((NPAD, D), jnp.float32),
        pltpu.SemaphoreType.DMA,
        pltpu.SemaphoreType.DMA,
        pltpu.SemaphoreType.DMA,
        pltpu.SemaphoreType.DMA,
    ],
)
def _scatter_kernel(src_hbm, dst_hbm, hp_hbm, out_hbm, si_v, di_v, rows0_v,
                    rows1_v, rows2_v, rows3_v, acc_sh, sem0, sem1, sem2, sem3):
    cid = lax.axis_index("c")
    sid = lax.axis_index("s")
    wid = sid * NC + cid
    rows = [rows0_v, rows1_v, rows2_v, rows3_v]
    sems = [sem0, sem1, sem2, sem3]

    def _zero(i, carry):
        for j in range(D // 16):
            rows0_v[i, pl.ds(j * 16, 16)] = jnp.zeros((16,), jnp.float32)
        return carry

    lax.fori_loop(0, CC, _zero, 0)

    r0 = sid * ROWS_PER_TILE
    for t in range(ROWS_PER_TILE // CC):
        pltpu.sync_copy(rows0_v, acc_sh.at[pl.ds(r0 + t * CC, CC)])
    plsc.subcore_barrier()

    for h in range(4):
        base = wid * ITERS_C + h * IBLK
        pltpu.sync_copy(src_hbm.at[pl.ds(base, IBLK)], si_v)
        pltpu.sync_copy(dst_hbm.at[pl.ds(base, IBLK)], di_v)

        for b in range(NBUF - 1):
            pltpu.async_copy(hp_hbm.at[si_v.at[b]], rows[b], sems[b])

        def _body(g, carry):
            for b in range(NBUF):
                jb = NBUF * g + b
                pltpu.make_async_copy(
                    hp_hbm.at[si_v.at[jb]], rows[b], sems[b]).wait()
                nb = (b + NBUF - 1) % NBUF

                @pl.when(jb + NBUF - 1 < IBLK)
                def _():
                    pltpu.async_copy(
                        hp_hbm.at[si_v.at[jb + NBUF - 1]], rows[nb], sems[nb])

                pltpu.sync_copy(rows[b], acc_sh.at[di_v.at[jb]], add=True)
            return carry

        lax.fori_loop(0, IBLK // NBUF, _body, 0)

    plsc.subcore_barrier()

    for t in range(ROWS_PER_TILE // CC):
        pltpu.sync_copy(acc_sh.at[pl.ds(r0 + t * CC, CC)], rows0_v)
        pltpu.sync_copy(rows0_v,
                        out_hbm.at[pl.ds(cid * NPAD + r0 + t * CC, CC)])


MB = 1000


def _mm_body(x_ref, w_ref, deg_ref, hp_ref, dis_ref):
    dis = lax.rsqrt(deg_ref[...])
    h = jnp.dot(x_ref[...], w_ref[...], preferred_element_type=jnp.float32)
    hp_ref[...] = h * dis
    dis_ref[...] = dis


_mm_call = pl.pallas_call(
    _mm_body,
    grid=(N // MB,),
    in_specs=[
        pl.BlockSpec((MB, D), lambda i: (i, 0)),
        pl.BlockSpec((D, D), lambda i: (0, 0)),
        pl.BlockSpec((MB, 1), lambda i: (i, 0)),
    ],
    out_specs=[
        pl.BlockSpec((MB, D), lambda i: (i, 0)),
        pl.BlockSpec((MB, 1), lambda i: (i, 0)),
    ],
    out_shape=[
        jax.ShapeDtypeStruct((N, D), jnp.float32),
        jax.ShapeDtypeStruct((N, 1), jnp.float32),
    ],
)


def _final_body(acc_ref, hp_ref, dis_ref, b_ref, o_ref):
    s = acc_ref[0] + acc_ref[1] + hp_ref[...]
    o_ref[...] = jnp.maximum(s * dis_ref[...] + b_ref[...], 0.0)


_final_call = pl.pallas_call(
    _final_body,
    grid=(N // MB,),
    in_specs=[
        pl.BlockSpec((NC, MB, D), lambda i: (0, i, 0)),
        pl.BlockSpec((MB, D), lambda i: (i, 0)),
        pl.BlockSpec((MB, 1), lambda i: (i, 0)),
        pl.BlockSpec((1, D), lambda i: (0, 0)),
    ],
    out_specs=pl.BlockSpec((MB, D), lambda i: (i, 0)),
    out_shape=jax.ShapeDtypeStruct((N, D), jnp.float32),
)


@jax.jit
def kernel(x, edge_index, W, b):
    src = edge_index[0].astype(jnp.int32)
    dst = edge_index[1].astype(jnp.int32)

    npad_e = EPAD - E
    pad_iota = jnp.arange(npad_e, dtype=jnp.int32)
    srcp = jnp.concatenate([src, pad_iota % N]).reshape(NCHUNK, CHUNK)
    dstp = jnp.concatenate(
        [dst, N + pad_iota % (NPAD - N)]).reshape(NCHUNK, CHUNK)

    degp = _deg_kernel(dstp)
    deg = (1.0 + degp[:N] + degp[NPAD:NPAD + N]).reshape(N, 1)

    hp, dis = _mm_call(x, W, deg)

    acc = _scatter_kernel(
        srcp.reshape(EPAD // CC, CC), dstp.reshape(EPAD // CC, CC),
        hp).reshape(NC, NPAD, D)

    return _final_call(acc, hp, dis, b.reshape(1, D))


# speedup vs baseline: 3.5750x; 1.0167x over previous
import functools

import jax
import jax.numpy as jnp
from jax import lax
from jax.experimental import pallas as pl
from jax.experimental.pallas import tpu as pltpu
from jax.experimental.pallas import tpu_sc as plsc

N = 10000
E = 320000
D = 128

NC, NS = 2, 16
NW = NC * NS
CHUNK = 128
NCHUNK = 2560
ITERS = NCHUNK // NW
EPAD = NCHUNK * CHUNK
NPAD = 10240
DEG_PER_TILE = NPAD // NS
ROWS_PER_TILE = NPAD // NS

_mesh = plsc.VectorSubcoreMesh(core_axis_name="c", subcore_axis_name="s")


@functools.partial(
    pl.kernel,
    out_type=jax.ShapeDtypeStruct((NC * NPAD,), jnp.float32),
    mesh=_mesh,
    scratch_types=[
        pltpu.VMEM((ITERS, CHUNK), jnp.int32),
        pltpu.VMEM((CHUNK,), jnp.float32),
        pltpu.VMEM((DEG_PER_TILE,), jnp.float32),
        pltpu.VMEM_SHARED((NPAD,), jnp.float32),
        pltpu.SemaphoreType.DMA,
    ],
)
def _deg_kernel(dst_hbm, out_hbm, di_v, ones_v, buf_v, acc_sh, sem):
    del sem
    cid = lax.axis_index("c")
    sid = lax.axis_index("s")
    wid = sid * NC + cid

    for j in range(CHUNK // 16):
        ones_v[pl.ds(j * 16, 16)] = jnp.ones((16,), jnp.float32)

    def _zero(i, carry):
        buf_v[pl.ds(i * 16, 16)] = jnp.zeros((16,), jnp.float32)
        return carry

    lax.fori_loop(0, DEG_PER_TILE // 16, _zero, 0)
    pltpu.sync_copy(buf_v, acc_sh.at[pl.ds(sid * DEG_PER_TILE, DEG_PER_TILE)])
    plsc.subcore_barrier()

    pltpu.sync_copy(dst_hbm.at[pl.ds(wid * ITERS, ITERS)], di_v)

    def _body(j, carry):
        pltpu.sync_copy(ones_v, acc_sh.at[di_v.at[j]], add=True)
        return carry

    lax.fori_loop(0, ITERS, _body, 0)
    plsc.subcore_barrier()

    pltpu.sync_copy(acc_sh.at[pl.ds(sid * DEG_PER_TILE, DEG_PER_TILE)], buf_v)
    pltpu.sync_copy(
        buf_v, out_hbm.at[pl.ds(cid * NPAD + sid * DEG_PER_TILE, DEG_PER_TILE)]
    )


CC = 64
ITERS_C = EPAD // CC // NW
IBLK = ITERS_C // 4
NBUF = 4


@functools.partial(
    pl.kernel,
    out_type=jax.ShapeDtypeStruct((NC * NPAD, D), jnp.float32),
    mesh=_mesh,
    scratch_types=[
        pltpu.VMEM((IBLK, CC), jnp.int32),
        pltpu.VMEM((IBLK, CC), jnp.int32),
        pltpu.VMEM((CC, D), jnp.float32),
        pltpu.VMEM((CC, D), jnp.float32),
        pltpu.VMEM((CC, D), jnp.float32),
        pltpu.VMEM((CC, D), jnp.float32),
        pltpu.VMEM_SHARED((NPAD, D), jnp.float32),
        pltpu.SemaphoreType.DMA,
        pltpu.SemaphoreType.DMA,
        pltpu.SemaphoreType.DMA,
        pltpu.SemaphoreType.DMA,
    ],
)
def _scatter_kernel(src_hbm, dst_hbm, hp_hbm, out_hbm, si_v, di_v, rows0_v,
                    rows1_v, rows2_v, rows3_v, acc_sh, sem0, sem1, sem2, sem3):
    cid = lax.axis_index("c")
    sid = lax.axis_index("s")
    wid = sid * NC + cid
    rows = [rows0_v, rows1_v, rows2_v, rows3_v]
    sems = [sem0, sem1, sem2, sem3]

    def _zero(i, carry):
        for j in range(D // 16):
            rows0_v[i, pl.ds(j * 16, 16)] = jnp.zeros((16,), jnp.float32)
        return carry

    lax.fori_loop(0, CC, _zero, 0)

    r0 = sid * ROWS_PER_TILE
    for t in range(ROWS_PER_TILE // CC):
        pltpu.sync_copy(rows0_v, acc_sh.at[pl.ds(r0 + t * CC, CC)])
    plsc.subcore_barrier()

    for h in range(4):
        base = wid * ITERS_C + h * IBLK
        pltpu.sync_copy(src_hbm.at[pl.ds(base, IBLK)], si_v)
        pltpu.sync_copy(dst_hbm.at[pl.ds(base, IBLK)], di_v)

        for b in range(NBUF - 1):
            pltpu.async_copy(hp_hbm.at[si_v.at[b]], rows[b], sems[b])

        def _body(g, carry):
            for b in range(NBUF):
                jb = NBUF * g + b
                pltpu.make_async_copy(
                    hp_hbm.at[si_v.at[jb]], rows[b], sems[b]).wait()
                nb = (b + NBUF - 1) % NBUF

                @pl.when(jb + NBUF - 1 < IBLK)
                def _():
                    pltpu.async_copy(
                        hp_hbm.at[si_v.at[jb + NBUF - 1]], rows[nb], sems[nb])

                pltpu.sync_copy(rows[b], acc_sh.at[di_v.at[jb]], add=True)
            return carry

        lax.fori_loop(0, IBLK // NBUF, _body, 0)

    plsc.subcore_barrier()

    for t in range(ROWS_PER_TILE // CC):
        pltpu.sync_copy(acc_sh.at[pl.ds(r0 + t * CC, CC)], rows0_v)
        pltpu.sync_copy(rows0_v,
                        out_hbm.at[pl.ds(cid * NPAD + r0 + t * CC, CC)])


def _mm_body(x_ref, w_ref, degp_ref, hp_ref, dis_ref):
    degp = degp_ref[...]
    deg = 1.0 + degp[:N] + degp[NPAD:NPAD + N]
    dis = lax.rsqrt(deg)
    h = jnp.dot(x_ref[...], w_ref[...], preferred_element_type=jnp.float32)
    hp_ref[...] = h * dis
    dis_ref[...] = dis


_mm_call = pl.pallas_call(
    _mm_body,
    out_shape=[
        jax.ShapeDtypeStruct((N, D), jnp.float32),
        jax.ShapeDtypeStruct((N, 1), jnp.float32),
    ],
)


def _final_body(acc_ref, hp_ref, dis_ref, b_ref, o_ref):
    acc = acc_ref[...]
    s = acc[0, :N] + acc[1, :N] + hp_ref[...]
    o_ref[...] = jnp.maximum(s * dis_ref[...] + b_ref[...], 0.0)


_final_call = pl.pallas_call(
    _final_body,
    out_shape=jax.ShapeDtypeStruct((N, D), jnp.float32),
)


@jax.jit
def kernel(x, edge_index, W, b):
    src = edge_index[0].astype(jnp.int32)
    dst = edge_index[1].astype(jnp.int32)

    npad_e = EPAD - E
    pad_iota = jnp.arange(npad_e, dtype=jnp.int32)
    srcp = jnp.concatenate([src, pad_iota % N]).reshape(NCHUNK, CHUNK)
    dstp = jnp.concatenate(
        [dst, N + pad_iota % (NPAD - N)]).reshape(NCHUNK, CHUNK)

    degp = _deg_kernel(dstp)

    hp, dis = _mm_call(x, W, degp.reshape(NC * NPAD, 1))

    acc = _scatter_kernel(
        srcp.reshape(EPAD // CC, CC), dstp.reshape(EPAD // CC, CC),
        hp).reshape(NC, NPAD, D)

    return _final_call(acc, hp, dis, b.reshape(1, D))


# speedup vs baseline: 3.5865x; 1.0032x over previous
import functools

import jax
import jax.numpy as jnp
from jax import lax
from jax.experimental import pallas as pl
from jax.experimental.pallas import tpu as pltpu
from jax.experimental.pallas import tpu_sc as plsc

N = 10000
E = 320000
D = 128

NC, NS = 2, 16
NW = NC * NS
CHUNK = 128
NCHUNK = 2560
ITERS = NCHUNK // NW
EPAD = NCHUNK * CHUNK
NPAD = 10240
DEG_PER_TILE = NPAD // NS
ROWS_PER_TILE = NPAD // NS

_mesh = plsc.VectorSubcoreMesh(core_axis_name="c", subcore_axis_name="s")


@functools.partial(
    pl.kernel,
    out_type=jax.ShapeDtypeStruct((NC * NPAD,), jnp.float32),
    mesh=_mesh,
    scratch_types=[
        pltpu.VMEM((ITERS, CHUNK), jnp.int32),
        pltpu.VMEM((CHUNK,), jnp.float32),
        pltpu.VMEM((DEG_PER_TILE,), jnp.float32),
        pltpu.VMEM_SHARED((NPAD,), jnp.float32),
        pltpu.SemaphoreType.DMA,
    ],
)
def _deg_kernel(dst_hbm, out_hbm, di_v, ones_v, buf_v, acc_sh, sem):
    del sem
    cid = lax.axis_index("c")
    sid = lax.axis_index("s")
    wid = sid * NC + cid

    for j in range(CHUNK // 16):
        ones_v[pl.ds(j * 16, 16)] = jnp.ones((16,), jnp.float32)

    def _zero(i, carry):
        buf_v[pl.ds(i * 16, 16)] = jnp.zeros((16,), jnp.float32)
        return carry

    lax.fori_loop(0, DEG_PER_TILE // 16, _zero, 0)
    pltpu.sync_copy(buf_v, acc_sh.at[pl.ds(sid * DEG_PER_TILE, DEG_PER_TILE)])
    plsc.subcore_barrier()

    pltpu.sync_copy(dst_hbm.at[pl.ds(wid * ITERS, ITERS)], di_v)

    def _body(j, carry):
        pltpu.sync_copy(ones_v, acc_sh.at[di_v.at[j]], add=True)
        return carry

    lax.fori_loop(0, ITERS, _body, 0)
    plsc.subcore_barrier()

    pltpu.sync_copy(
        acc_sh.at[pl.ds(sid * DEG_PER_TILE, DEG_PER_TILE)],
        out_hbm.at[pl.ds(cid * NPAD + sid * DEG_PER_TILE, DEG_PER_TILE)],
    )


CC = 64
ITERS_C = EPAD // CC // NW
IBLK = ITERS_C // 4
NBUF = 4


@functools.partial(
    pl.kernel,
    out_type=jax.ShapeDtypeStruct((NC * NPAD, D), jnp.float32),
    mesh=_mesh,
    scratch_types=[
        pltpu.VMEM((IBLK, CC), jnp.int32),
        pltpu.VMEM((IBLK, CC), jnp.int32),
        pltpu.VMEM((CC, D), jnp.float32),
        pltpu.VMEM((CC, D), jnp.float32),
        pltpu.VMEM((CC, D), jnp.float32),
        pltpu.VMEM((CC, D), jnp.float32),
        pltpu.VMEM_SHARED((NPAD, D), jnp.float32),
        pltpu.SemaphoreType.DMA,
        pltpu.SemaphoreType.DMA,
        pltpu.SemaphoreType.DMA,
        pltpu.SemaphoreType.DMA,
    ],
)
def _scatter_kernel(src_hbm, dst_hbm, hp_hbm, out_hbm, si_v, di_v, rows0_v,
                    rows1_v, rows2_v, rows3_v, acc_sh, sem0, sem1, sem2, sem3):
    cid = lax.axis_index("c")
    sid = lax.axis_index("s")
    wid = sid * NC + cid
    rows = [rows0_v, rows1_v, rows2_v, rows3_v]
    sems = [sem0, sem1, sem2, sem3]

    def _zero(i, carry):
        for j in range(D // 16):
            rows0_v[i, pl.ds(j * 16, 16)] = jnp.zeros((16,), jnp.float32)
        return carry

    lax.fori_loop(0, CC, _zero, 0)

    r0 = sid * ROWS_PER_TILE
    for t in range(ROWS_PER_TILE // CC):
        pltpu.sync_copy(rows0_v, acc_sh.at[pl.ds(r0 + t * CC, CC)])
    plsc.subcore_barrier()

    for h in range(4):
        base = wid * ITERS_C + h * IBLK
        pltpu.sync_copy(src_hbm.at[pl.ds(base, IBLK)], si_v)
        pltpu.sync_copy(dst_hbm.at[pl.ds(base, IBLK)], di_v)

        for b in range(NBUF - 1):
            pltpu.async_copy(hp_hbm.at[si_v.at[b]], rows[b], sems[b])

        def _body(g, carry):
            for b in range(NBUF):
                jb = NBUF * g + b
                pltpu.make_async_copy(
                    hp_hbm.at[si_v.at[jb]], rows[b], sems[b]).wait()
                nb = (b + NBUF - 1) % NBUF

                @pl.when(jb + NBUF - 1 < IBLK)
                def _():
                    pltpu.async_copy(
                        hp_hbm.at[si_v.at[jb + NBUF - 1]], rows[nb], sems[nb])

                pltpu.sync_copy(rows[b], acc_sh.at[di_v.at[jb]], add=True)
            return carry

        lax.fori_loop(0, IBLK // NBUF, _body, 0)

    plsc.subcore_barrier()

    pltpu.sync_copy(acc_sh.at[pl.ds(r0, ROWS_PER_TILE)],
                    out_hbm.at[pl.ds(cid * NPAD + r0, ROWS_PER_TILE)])


def _mm_body(x_ref, w_ref, degp_ref, hp_ref, dis_ref):
    degp = degp_ref[...]
    deg = 1.0 + degp[:N] + degp[NPAD:NPAD + N]
    dis = lax.rsqrt(deg)
    h = jnp.dot(x_ref[...], w_ref[...], preferred_element_type=jnp.float32)
    hp_ref[...] = h * dis
    dis_ref[...] = dis


_mm_call = pl.pallas_call(
    _mm_body,
    out_shape=[
        jax.ShapeDtypeStruct((N, D), jnp.float32),
        jax.ShapeDtypeStruct((N, 1), jnp.float32),
    ],
)


def _final_body(acc_ref, hp_ref, dis_ref, b_ref, o_ref):
    acc = acc_ref[...]
    s = acc[0, :N] + acc[1, :N] + hp_ref[...]
    o_ref[...] = jnp.maximum(s * dis_ref[...] + b_ref[...], 0.0)


_final_call = pl.pallas_call(
    _final_body,
    out_shape=jax.ShapeDtypeStruct((N, D), jnp.float32),
)


@jax.jit
def kernel(x, edge_index, W, b):
    src = edge_index[0].astype(jnp.int32)
    dst = edge_index[1].astype(jnp.int32)

    npad_e = EPAD - E
    pad_iota = jnp.arange(npad_e, dtype=jnp.int32)
    srcp = jnp.concatenate([src, pad_iota % N]).reshape(NCHUNK, CHUNK)
    dstp = jnp.concatenate(
        [dst, N + pad_iota % (NPAD - N)]).reshape(NCHUNK, CHUNK)

    degp = _deg_kernel(dstp)

    hp, dis = _mm_call(x, W, degp.reshape(NC * NPAD, 1))

    acc = _scatter_kernel(
        srcp.reshape(EPAD // CC, CC), dstp.reshape(EPAD // CC, CC),
        hp).reshape(NC, NPAD, D)

    return _final_call(acc, hp, dis, b.reshape(1, D))
